# Initial kernel scaffold; baseline (speedup 1.0000x reference)
import jax
import jax.numpy as jnp
from jax.experimental import pallas as pl


def kernel(x, edge_index, W1, b1, W2, b2, fc_W, fc_b):
    raise NotImplementedError("write your pallas kernel here")


# speedup vs baseline: 57.4048x; 57.4048x over previous
import functools

import jax
import jax.numpy as jnp
from jax import lax
from jax.experimental import pallas as pl
from jax.experimental.pallas import tpu as pltpu
from jax.experimental.pallas import tpu_sc as plsc

_NC = 2
_NS = 16
_CH = 800
_CHD = 2000


def _fill_1d(ref, n, val):
    def body(i, c):
        ref[pl.ds(i * 16, 16)] = jnp.full((16,), val, ref.dtype)
        return c
    lax.fori_loop(0, n // 16, body, 0)


def _make_deg(E, NPAD):
    EC = E // _NC
    ET = EC // _NS
    NIT = ET // _CHD
    PT = NPAD // _NS
    mesh = plsc.VectorSubcoreMesh(core_axis_name="c", subcore_axis_name="s")

    @functools.partial(
        pl.kernel,
        out_type=(jax.ShapeDtypeStruct((NPAD,), jnp.float32),
                  jax.ShapeDtypeStruct((NPAD,), jnp.float32)),
        mesh=mesh,
        scratch_types=[
            pltpu.VMEM((_CHD,), jnp.int32),
            pltpu.VMEM((_CHD,), jnp.float32),
            pltpu.VMEM((PT,), jnp.float32),
            pltpu.VMEM_SHARED((NPAD,), jnp.float32),
        ],
        compiler_params=pltpu.CompilerParams(use_tc_tiling_on_sc=False),
    )
    def deg(dst_hbm, cnt0_hbm, cnt1_hbm, didx, ones, zeros, acc):
        c = lax.axis_index("c")
        s = lax.axis_index("s")
        _fill_1d(ones, _CHD, 1.0)
        _fill_1d(zeros, PT, 0.0)
        pltpu.sync_copy(zeros, acc.at[pl.ds(s * PT, PT)])
        plsc.subcore_barrier()
        tbase = c * EC + s * ET

        def body(i, carry):
            pltpu.sync_copy(dst_hbm.at[pl.ds(tbase + i * _CHD, _CHD)], didx)
            pltpu.sync_copy(ones, acc.at[didx], add=True)
            return carry

        lax.fori_loop(0, NIT, body, 0)
        plsc.subcore_barrier()
        pltpu.sync_copy(acc.at[pl.ds(s * PT, PT)], zeros)

        @pl.when(c == 0)
        def _():
            pltpu.sync_copy(zeros, cnt0_hbm.at[pl.ds(s * PT, PT)])

        @pl.when(c == 1)
        def _():
            pltpu.sync_copy(zeros, cnt1_hbm.at[pl.ds(s * PT, PT)])

    return deg


def _make_scatter(E, NPAD, F):
    EC = E // _NC
    ET = EC // _NS
    NIT = ET // _CH
    PT = NPAD // _NS
    ZR = PT // 17
    mesh = plsc.VectorSubcoreMesh(core_axis_name="c", subcore_axis_name="s")

    @functools.partial(
        pl.kernel,
        out_type=(jax.ShapeDtypeStruct((NPAD, F), jnp.float32),
                  jax.ShapeDtypeStruct((NPAD, F), jnp.float32)),
        mesh=mesh,
        scratch_types=[
            pltpu.VMEM((_CH,), jnp.int32),
            pltpu.VMEM((_CH,), jnp.int32),
            pltpu.VMEM((_CH, F), jnp.float32),
            pltpu.VMEM((ZR, F), jnp.float32),
            pltpu.VMEM_SHARED((NPAD, F), jnp.float32),
            pltpu.SemaphoreType.DMA,
        ],
        compiler_params=pltpu.CompilerParams(use_tc_tiling_on_sc=False),
    )
    def scat(src_hbm, dst_hbm, v_hbm, acc0_hbm, acc1_hbm,
             sidx, didx, rows, zrows, acc, gsem):
        c = lax.axis_index("c")
        s = lax.axis_index("s")

        def zb(i, carry):
            zrows[i, :] = jnp.zeros((16,), jnp.float32)
            return carry

        lax.fori_loop(0, ZR, zb, 0)
        for j in range(PT // ZR):
            pltpu.sync_copy(zrows, acc.at[pl.ds(s * PT + j * ZR, ZR)])
        plsc.subcore_barrier()
        tbase = c * EC + s * ET

        def body(i, carry):
            b = tbase + i * _CH
            pltpu.sync_copy(src_hbm.at[pl.ds(b, _CH)], sidx)
            pltpu.sync_copy(dst_hbm.at[pl.ds(b, _CH)], didx)
            pltpu.async_copy(v_hbm.at[sidx], rows, gsem).wait()
            pltpu.sync_copy(rows, acc.at[didx], add=True)
            return carry

        lax.fori_loop(0, NIT, body, 0)
        plsc.subcore_barrier()
        for j in range(PT // ZR):
            pltpu.sync_copy(acc.at[pl.ds(s * PT + j * ZR, ZR)], zrows)

            @pl.when(c == 0)
            def _():
                pltpu.sync_copy(zrows, acc0_hbm.at[pl.ds(s * PT + j * ZR, ZR)])

            @pl.when(c == 1)
            def _():
                pltpu.sync_copy(zrows, acc1_hbm.at[pl.ds(s * PT + j * ZR, ZR)])

    return scat


def _t1(cnt0, cnt1, x, W1, NPAD):
    BR = 3128
    NG = NPAD // BR

    def body(c0, c1, xr, w1, y1_ref, dinv_ref):
        deg = c0[...] + c1[...] + 1.0
        dinv = lax.rsqrt(deg)
        dinv_ref[...] = dinv
        y1_ref[...] = jnp.dot(xr[...] * dinv, w1[...],
                              preferred_element_type=jnp.float32)

    return pl.pallas_call(
        body,
        grid=(NG,),
        in_specs=[
            pl.BlockSpec((BR, 1), lambda i: (i, 0)),
            pl.BlockSpec((BR, 1), lambda i: (i, 0)),
            pl.BlockSpec((BR, 2), lambda i: (i, 0)),
            pl.BlockSpec((2, 16), lambda i: (0, 0)),
        ],
        out_specs=(pl.BlockSpec((BR, 16), lambda i: (i, 0)),
                   pl.BlockSpec((BR, 1), lambda i: (i, 0))),
        out_shape=(jax.ShapeDtypeStruct((NPAD, 16), jnp.float32),
                   jax.ShapeDtypeStruct((NPAD, 1), jnp.float32)),
    )(cnt0, cnt1, x, W1)


def _t2(acc0, acc1, y1, dinv, b1, W2, NPAD):
    BR = 3128
    NG = NPAD // BR

    def body(a0_ref, a1_ref, y1_ref, dinv_ref, b1_ref, w2_ref, y2_ref):
        ssum = a0_ref[...] + a1_ref[...] + y1_ref[...]
        h1 = jnp.maximum(dinv_ref[...] * ssum + b1_ref[...], 0.0)
        y2_ref[...] = dinv_ref[...] * jnp.dot(
            h1, w2_ref[...], preferred_element_type=jnp.float32)

    return pl.pallas_call(
        body,
        grid=(NG,),
        in_specs=[
            pl.BlockSpec((BR, 16), lambda i: (i, 0)),
            pl.BlockSpec((BR, 16), lambda i: (i, 0)),
            pl.BlockSpec((BR, 16), lambda i: (i, 0)),
            pl.BlockSpec((BR, 1), lambda i: (i, 0)),
            pl.BlockSpec((1, 16), lambda i: (0, 0)),
            pl.BlockSpec((16, 16), lambda i: (0, 0)),
        ],
        out_specs=pl.BlockSpec((BR, 16), lambda i: (i, 0)),
        out_shape=jax.ShapeDtypeStruct((NPAD, 16), jnp.float32),
    )(acc0, acc1, y1, dinv, b1, W2)


def _t3(acc0, acc1, y2, dinv, b2, fc_W, fc_b, N):
    NPAD = acc0.shape[0]
    BR = 3128
    NG = NPAD // BR

    def body(a0_ref, a1_ref, y2_ref, dinv_ref, b2_ref, fcw_ref, fcb_ref,
             out_ref, accum):
        i = pl.program_id(0)

        @pl.when(i == 0)
        def _():
            accum[...] = jnp.zeros_like(accum)

        ssum = a0_ref[...] + a1_ref[...] + y2_ref[...]
        h2 = jnp.maximum(dinv_ref[...] * ssum + b2_ref[...], 0.0)
        rid = lax.broadcasted_iota(jnp.int32, (BR, 16), 0) + i * BR
        h2 = jnp.where(rid < N, h2, 0.0)
        accum[...] += jnp.sum(h2, axis=0, keepdims=True)

        @pl.when(i == NG - 1)
        def _():
            m = accum[...] * (1.0 / N)
            logits = jnp.dot(m, fcw_ref[...],
                             preferred_element_type=jnp.float32) + fcb_ref[...]
            z = logits - jnp.max(logits, axis=1, keepdims=True)
            out_ref[...] = z - jnp.log(
                jnp.sum(jnp.exp(z), axis=1, keepdims=True))

    return pl.pallas_call(
        body,
        grid=(NG,),
        in_specs=[
            pl.BlockSpec((BR, 16), lambda i: (i, 0)),
            pl.BlockSpec((BR, 16), lambda i: (i, 0)),
            pl.BlockSpec((BR, 16), lambda i: (i, 0)),
            pl.BlockSpec((BR, 1), lambda i: (i, 0)),
            pl.BlockSpec((1, 16), lambda i: (0, 0)),
            pl.BlockSpec((16, fc_W.shape[1]), lambda i: (0, 0)),
            pl.BlockSpec((1, fc_W.shape[1]), lambda i: (0, 0)),
        ],
        out_specs=pl.BlockSpec((1, fc_W.shape[1]), lambda i: (0, 0)),
        out_shape=jax.ShapeDtypeStruct((1, fc_W.shape[1]), jnp.float32),
        scratch_shapes=[pltpu.VMEM((1, 16), jnp.float32)],
    )(acc0, acc1, y2, dinv, b2, fc_W, fc_b)


def kernel(x, edge_index, W1, b1, W2, b2, fc_W, fc_b):
    N, E = x.shape[0], edge_index.shape[1]
    NPAD = ((N + _NS * 16 - 1) // (_NS * 16)) * (_NS * 16)

    deg_call = _make_deg(E, NPAD)
    scat_call = _make_scatter(E, NPAD, 16)

    e_src = edge_index[0]
    e_dst = edge_index[1]
    cnt_a, cnt_b = deg_call(e_dst)
    cnt0 = cnt_a.reshape(NPAD, 1)
    cnt1 = cnt_b.reshape(NPAD, 1)
    xp = jnp.concatenate(
        [x, jnp.zeros((NPAD - N, x.shape[1]), x.dtype)], axis=0)
    b1r = b1.reshape(1, 16)
    b2r = b2.reshape(1, 16)
    fcbr = fc_b.reshape(1, fc_b.shape[0])

    y1, dinv = _t1(cnt0, cnt1, xp, W1, NPAD)
    s1a, s1b = scat_call(e_src, e_dst, y1)
    y2 = _t2(s1a, s1b, y1, dinv, b1r, W2, NPAD)
    s2a, s2b = scat_call(e_src, e_dst, y2)
    return _t3(s2a, s2b, y2, dinv, b2r, fc_W, fcbr, N)


# speedup vs baseline: 60.8230x; 1.0595x over previous
import functools

import jax
import jax.numpy as jnp
from jax import lax
from jax.experimental import pallas as pl
from jax.experimental.pallas import tpu as pltpu
from jax.experimental.pallas import tpu_sc as plsc

_NC = 2
_NS = 16
_CH = 1000
_CHD = 2000


def _fill_1d(ref, n, val):
    def body(i, c):
        ref[pl.ds(i * 16, 16)] = jnp.full((16,), val, ref.dtype)
        return c
    lax.fori_loop(0, n // 16, body, 0)


def _make_deg(E, NPAD):
    EC = E // _NC
    ET = EC // _NS
    NIT = ET // _CHD
    PT = NPAD // _NS
    mesh = plsc.VectorSubcoreMesh(core_axis_name="c", subcore_axis_name="s")

    @functools.partial(
        pl.kernel,
        out_type=(jax.ShapeDtypeStruct((NPAD,), jnp.float32),
                  jax.ShapeDtypeStruct((NPAD,), jnp.float32)),
        mesh=mesh,
        scratch_types=[
            pltpu.VMEM((_CHD,), jnp.int32),
            pltpu.VMEM((_CHD,), jnp.float32),
            pltpu.VMEM((PT,), jnp.float32),
            pltpu.VMEM_SHARED((NPAD,), jnp.float32),
        ],
        compiler_params=pltpu.CompilerParams(use_tc_tiling_on_sc=False),
    )
    def deg(dst_hbm, cnt0_hbm, cnt1_hbm, didx, ones, zeros, acc):
        c = lax.axis_index("c")
        s = lax.axis_index("s")
        _fill_1d(ones, _CHD, 1.0)
        _fill_1d(zeros, PT, 0.0)
        pltpu.sync_copy(zeros, acc.at[pl.ds(s * PT, PT)])
        plsc.subcore_barrier()
        tbase = c * EC + s * ET

        def body(i, carry):
            pltpu.sync_copy(dst_hbm.at[pl.ds(tbase + i * _CHD, _CHD)], didx)
            pltpu.sync_copy(ones, acc.at[didx], add=True)
            return carry

        lax.fori_loop(0, NIT, body, 0)
        plsc.subcore_barrier()
        pltpu.sync_copy(acc.at[pl.ds(s * PT, PT)], zeros)

        @pl.when(c == 0)
        def _():
            pltpu.sync_copy(zeros, cnt0_hbm.at[pl.ds(s * PT, PT)])

        @pl.when(c == 1)
        def _():
            pltpu.sync_copy(zeros, cnt1_hbm.at[pl.ds(s * PT, PT)])

    return deg


def _make_scatter(E, NPAD, F):
    EC = E // _NC
    ET = EC // _NS
    NIT = ET // _CH
    PT = NPAD // _NS
    ZR = PT // 17
    mesh = plsc.VectorSubcoreMesh(core_axis_name="c", subcore_axis_name="s")

    @functools.partial(
        pl.kernel,
        out_type=(jax.ShapeDtypeStruct((NPAD, F), jnp.float32),
                  jax.ShapeDtypeStruct((NPAD, F), jnp.float32)),
        mesh=mesh,
        scratch_types=[
            pltpu.VMEM((_CH,), jnp.int32),
            pltpu.VMEM((_CH,), jnp.int32),
            pltpu.VMEM((_CH, F), jnp.float32),
            pltpu.VMEM((ZR, F), jnp.float32),
            pltpu.VMEM_SHARED((NPAD, F), jnp.float32),
            pltpu.SemaphoreType.DMA,
        ],
        compiler_params=pltpu.CompilerParams(use_tc_tiling_on_sc=False),
    )
    def scat(src_hbm, dst_hbm, v_hbm, acc0_hbm, acc1_hbm,
             sidx, didx, rows, zrows, acc, gsem):
        c = lax.axis_index("c")
        s = lax.axis_index("s")

        def zb(i, carry):
            zrows[i, :] = jnp.zeros((16,), jnp.float32)
            return carry

        lax.fori_loop(0, ZR, zb, 0)
        for j in range(PT // ZR):
            pltpu.sync_copy(zrows, acc.at[pl.ds(s * PT + j * ZR, ZR)])
        plsc.subcore_barrier()
        tbase = c * EC + s * ET

        def body(i, carry):
            b = tbase + i * _CH
            pltpu.sync_copy(src_hbm.at[pl.ds(b, _CH)], sidx)
            pltpu.sync_copy(dst_hbm.at[pl.ds(b, _CH)], didx)
            pltpu.async_copy(v_hbm.at[sidx], rows, gsem).wait()
            pltpu.sync_copy(rows, acc.at[didx], add=True)
            return carry

        lax.fori_loop(0, NIT, body, 0)
        plsc.subcore_barrier()
        for j in range(PT // ZR):
            pltpu.sync_copy(acc.at[pl.ds(s * PT + j * ZR, ZR)], zrows)

            @pl.when(c == 0)
            def _():
                pltpu.sync_copy(zrows, acc0_hbm.at[pl.ds(s * PT + j * ZR, ZR)])

            @pl.when(c == 1)
            def _():
                pltpu.sync_copy(zrows, acc1_hbm.at[pl.ds(s * PT + j * ZR, ZR)])

    return scat


def _t1(cnt0, cnt1, x, W1, NPAD):
    BR = 3128
    NG = NPAD // BR

    def body(c0, c1, xr, w1, y1_ref, dinv_ref):
        deg = c0[...] + c1[...] + 1.0
        dinv = lax.rsqrt(deg)
        dinv_ref[...] = dinv
        y1_ref[...] = jnp.dot(xr[...] * dinv, w1[...],
                              preferred_element_type=jnp.float32)

    return pl.pallas_call(
        body,
        grid=(NG,),
        in_specs=[
            pl.BlockSpec((BR, 1), lambda i: (i, 0)),
            pl.BlockSpec((BR, 1), lambda i: (i, 0)),
            pl.BlockSpec((BR, 2), lambda i: (i, 0)),
            pl.BlockSpec((2, 16), lambda i: (0, 0)),
        ],
        out_specs=(pl.BlockSpec((BR, 16), lambda i: (i, 0)),
                   pl.BlockSpec((BR, 1), lambda i: (i, 0))),
        out_shape=(jax.ShapeDtypeStruct((NPAD, 16), jnp.float32),
                   jax.ShapeDtypeStruct((NPAD, 1), jnp.float32)),
    )(cnt0, cnt1, x, W1)


def _t2(acc0, acc1, y1, dinv, b1, W2, NPAD):
    BR = 3128
    NG = NPAD // BR

    def body(a0_ref, a1_ref, y1_ref, dinv_ref, b1_ref, w2_ref, y2_ref):
        ssum = a0_ref[...] + a1_ref[...] + y1_ref[...]
        h1 = jnp.maximum(dinv_ref[...] * ssum + b1_ref[...], 0.0)
        y2_ref[...] = dinv_ref[...] * jnp.dot(
            h1, w2_ref[...], preferred_element_type=jnp.float32)

    return pl.pallas_call(
        body,
        grid=(NG,),
        in_specs=[
            pl.BlockSpec((BR, 16), lambda i: (i, 0)),
            pl.BlockSpec((BR, 16), lambda i: (i, 0)),
            pl.BlockSpec((BR, 16), lambda i: (i, 0)),
            pl.BlockSpec((BR, 1), lambda i: (i, 0)),
            pl.BlockSpec((1, 16), lambda i: (0, 0)),
            pl.BlockSpec((16, 16), lambda i: (0, 0)),
        ],
        out_specs=pl.BlockSpec((BR, 16), lambda i: (i, 0)),
        out_shape=jax.ShapeDtypeStruct((NPAD, 16), jnp.float32),
    )(acc0, acc1, y1, dinv, b1, W2)


def _t3(acc0, acc1, y2, dinv, b2, fc_W, fc_b, N):
    NPAD = acc0.shape[0]
    BR = 3128
    NG = NPAD // BR

    def body(a0_ref, a1_ref, y2_ref, dinv_ref, b2_ref, fcw_ref, fcb_ref,
             out_ref, accum):
        i = pl.program_id(0)

        @pl.when(i == 0)
        def _():
            accum[...] = jnp.zeros_like(accum)

        ssum = a0_ref[...] + a1_ref[...] + y2_ref[...]
        h2 = jnp.maximum(dinv_ref[...] * ssum + b2_ref[...], 0.0)
        rid = lax.broadcasted_iota(jnp.int32, (BR, 16), 0) + i * BR
        h2 = jnp.where(rid < N, h2, 0.0)
        accum[...] += jnp.sum(h2, axis=0, keepdims=True)

        @pl.when(i == NG - 1)
        def _():
            m = accum[...] * (1.0 / N)
            logits = jnp.dot(m, fcw_ref[...],
                             preferred_element_type=jnp.float32) + fcb_ref[...]
            z = logits - jnp.max(logits, axis=1, keepdims=True)
            out_ref[...] = z - jnp.log(
                jnp.sum(jnp.exp(z), axis=1, keepdims=True))

    return pl.pallas_call(
        body,
        grid=(NG,),
        in_specs=[
            pl.BlockSpec((BR, 16), lambda i: (i, 0)),
            pl.BlockSpec((BR, 16), lambda i: (i, 0)),
            pl.BlockSpec((BR, 16), lambda i: (i, 0)),
            pl.BlockSpec((BR, 1), lambda i: (i, 0)),
            pl.BlockSpec((1, 16), lambda i: (0, 0)),
            pl.BlockSpec((16, fc_W.shape[1]), lambda i: (0, 0)),
            pl.BlockSpec((1, fc_W.shape[1]), lambda i: (0, 0)),
        ],
        out_specs=pl.BlockSpec((1, fc_W.shape[1]), lambda i: (0, 0)),
        out_shape=jax.ShapeDtypeStruct((1, fc_W.shape[1]), jnp.float32),
        scratch_shapes=[pltpu.VMEM((1, 16), jnp.float32)],
    )(acc0, acc1, y2, dinv, b2, fc_W, fc_b)


def kernel(x, edge_index, W1, b1, W2, b2, fc_W, fc_b):
    N, E = x.shape[0], edge_index.shape[1]
    NPAD = ((N + _NS * 16 - 1) // (_NS * 16)) * (_NS * 16)

    deg_call = _make_deg(E, NPAD)
    scat_call = _make_scatter(E, NPAD, 16)

    e_src = edge_index[0]
    e_dst = edge_index[1]
    cnt_a, cnt_b = deg_call(e_dst)
    cnt0 = cnt_a.reshape(NPAD, 1)
    cnt1 = cnt_b.reshape(NPAD, 1)
    xp = jnp.concatenate(
        [x, jnp.zeros((NPAD - N, x.shape[1]), x.dtype)], axis=0)
    b1r = b1.reshape(1, 16)
    b2r = b2.reshape(1, 16)
    fcbr = fc_b.reshape(1, fc_b.shape[0])

    y1, dinv = _t1(cnt0, cnt1, xp, W1, NPAD)
    s1a, s1b = scat_call(e_src, e_dst, y1)
    y2 = _t2(s1a, s1b, y1, dinv, b1r, W2, NPAD)
    s2a, s2b = scat_call(e_src, e_dst, y2)
    return _t3(s2a, s2b, y2, dinv, b2r, fc_W, fcbr, N)


# speedup vs baseline: 61.6021x; 1.0128x over previous
import functools

import jax
import jax.numpy as jnp
from jax import lax
from jax.experimental import pallas as pl
from jax.experimental.pallas import tpu as pltpu
from jax.experimental.pallas import tpu_sc as plsc

_NC = 2
_NS = 16
_CH = 1000
_CHD = 4000


def _fill_1d(ref, n, val):
    def body(i, c):
        ref[pl.ds(i * 16, 16)] = jnp.full((16,), val, ref.dtype)
        return c
    lax.fori_loop(0, n // 16, body, 0)


def _make_deg(E, NPAD):
    EC = E // _NC
    ET = EC // _NS
    NIT = ET // _CHD
    PT = NPAD // _NS
    mesh = plsc.VectorSubcoreMesh(core_axis_name="c", subcore_axis_name="s")

    @functools.partial(
        pl.kernel,
        out_type=(jax.ShapeDtypeStruct((NPAD,), jnp.float32),
                  jax.ShapeDtypeStruct((NPAD,), jnp.float32)),
        mesh=mesh,
        scratch_types=[
            pltpu.VMEM((_CHD,), jnp.int32),
            pltpu.VMEM((_CHD,), jnp.float32),
            pltpu.VMEM((PT,), jnp.float32),
            pltpu.VMEM_SHARED((NPAD,), jnp.float32),
        ],
        compiler_params=pltpu.CompilerParams(use_tc_tiling_on_sc=False),
    )
    def deg(dst_hbm, cnt0_hbm, cnt1_hbm, didx, ones, zeros, acc):
        c = lax.axis_index("c")
        s = lax.axis_index("s")
        _fill_1d(ones, _CHD, 1.0)
        _fill_1d(zeros, PT, 0.0)
        pltpu.sync_copy(zeros, acc.at[pl.ds(s * PT, PT)])
        plsc.subcore_barrier()
        tbase = c * EC + s * ET

        def body(i, carry):
            pltpu.sync_copy(dst_hbm.at[pl.ds(tbase + i * _CHD, _CHD)], didx)
            pltpu.sync_copy(ones, acc.at[didx], add=True)
            return carry

        lax.fori_loop(0, NIT, body, 0)
        plsc.subcore_barrier()
        pltpu.sync_copy(acc.at[pl.ds(s * PT, PT)], zeros)

        @pl.when(c == 0)
        def _():
            pltpu.sync_copy(zeros, cnt0_hbm.at[pl.ds(s * PT, PT)])

        @pl.when(c == 1)
        def _():
            pltpu.sync_copy(zeros, cnt1_hbm.at[pl.ds(s * PT, PT)])

    return deg


def _make_scatter(E, NPAD, F):
    EC = E // _NC
    ET = EC // _NS
    NIT = ET // _CH
    PT = NPAD // _NS
    ZR = PT // 17
    mesh = plsc.VectorSubcoreMesh(core_axis_name="c", subcore_axis_name="s")

    @functools.partial(
        pl.kernel,
        out_type=(jax.ShapeDtypeStruct((NPAD, F), jnp.float32),
                  jax.ShapeDtypeStruct((NPAD, F), jnp.float32)),
        mesh=mesh,
        scratch_types=[
            pltpu.VMEM((_CH,), jnp.int32),
            pltpu.VMEM((_CH,), jnp.int32),
            pltpu.VMEM((_CH, F), jnp.float32),
            pltpu.VMEM((ZR, F), jnp.float32),
            pltpu.VMEM_SHARED((NPAD, F), jnp.float32),
            pltpu.SemaphoreType.DMA,
        ],
        compiler_params=pltpu.CompilerParams(use_tc_tiling_on_sc=False),
    )
    def scat(src_hbm, dst_hbm, v_hbm, acc0_hbm, acc1_hbm,
             sidx, didx, rows, zrows, acc, gsem):
        c = lax.axis_index("c")
        s = lax.axis_index("s")

        def zb(i, carry):
            zrows[i, :] = jnp.zeros((16,), jnp.float32)
            return carry

        lax.fori_loop(0, ZR, zb, 0)
        for j in range(PT // ZR):
            pltpu.sync_copy(zrows, acc.at[pl.ds(s * PT + j * ZR, ZR)])
        plsc.subcore_barrier()
        tbase = c * EC + s * ET

        def body(i, carry):
            b = tbase + i * _CH
            pltpu.sync_copy(src_hbm.at[pl.ds(b, _CH)], sidx)
            pltpu.sync_copy(dst_hbm.at[pl.ds(b, _CH)], didx)
            pltpu.async_copy(v_hbm.at[sidx], rows, gsem).wait()
            pltpu.sync_copy(rows, acc.at[didx], add=True)
            return carry

        lax.fori_loop(0, NIT, body, 0)
        plsc.subcore_barrier()
        for j in range(PT // ZR):
            pltpu.sync_copy(acc.at[pl.ds(s * PT + j * ZR, ZR)], zrows)

            @pl.when(c == 0)
            def _():
                pltpu.sync_copy(zrows, acc0_hbm.at[pl.ds(s * PT + j * ZR, ZR)])

            @pl.when(c == 1)
            def _():
                pltpu.sync_copy(zrows, acc1_hbm.at[pl.ds(s * PT + j * ZR, ZR)])

    return scat


def _t1(cnt0, cnt1, x, W1, NPAD):
    BR = 3128
    NG = NPAD // BR

    def body(c0, c1, xr, w1, y1_ref, dinv_ref):
        deg = c0[...] + c1[...] + 1.0
        dinv = lax.rsqrt(deg)
        dinv_ref[...] = dinv
        y1_ref[...] = jnp.dot(xr[...] * dinv, w1[...],
                              preferred_element_type=jnp.float32)

    return pl.pallas_call(
        body,
        grid=(NG,),
        in_specs=[
            pl.BlockSpec((BR, 1), lambda i: (i, 0)),
            pl.BlockSpec((BR, 1), lambda i: (i, 0)),
            pl.BlockSpec((BR, 2), lambda i: (i, 0)),
            pl.BlockSpec((2, 16), lambda i: (0, 0)),
        ],
        out_specs=(pl.BlockSpec((BR, 16), lambda i: (i, 0)),
                   pl.BlockSpec((BR, 1), lambda i: (i, 0))),
        out_shape=(jax.ShapeDtypeStruct((NPAD, 16), jnp.float32),
                   jax.ShapeDtypeStruct((NPAD, 1), jnp.float32)),
    )(cnt0, cnt1, x, W1)


def _t2(acc0, acc1, y1, dinv, b1, W2, NPAD):
    BR = 3128
    NG = NPAD // BR

    def body(a0_ref, a1_ref, y1_ref, dinv_ref, b1_ref, w2_ref, y2_ref):
        ssum = a0_ref[...] + a1_ref[...] + y1_ref[...]
        h1 = jnp.maximum(dinv_ref[...] * ssum + b1_ref[...], 0.0)
        y2_ref[...] = dinv_ref[...] * jnp.dot(
            h1, w2_ref[...], preferred_element_type=jnp.float32)

    return pl.pallas_call(
        body,
        grid=(NG,),
        in_specs=[
            pl.BlockSpec((BR, 16), lambda i: (i, 0)),
            pl.BlockSpec((BR, 16), lambda i: (i, 0)),
            pl.BlockSpec((BR, 16), lambda i: (i, 0)),
            pl.BlockSpec((BR, 1), lambda i: (i, 0)),
            pl.BlockSpec((1, 16), lambda i: (0, 0)),
            pl.BlockSpec((16, 16), lambda i: (0, 0)),
        ],
        out_specs=pl.BlockSpec((BR, 16), lambda i: (i, 0)),
        out_shape=jax.ShapeDtypeStruct((NPAD, 16), jnp.float32),
    )(acc0, acc1, y1, dinv, b1, W2)


def _t3(acc0, acc1, y2, dinv, b2, fc_W, fc_b, N):
    NPAD = acc0.shape[0]
    BR = 3128
    NG = NPAD // BR

    def body(a0_ref, a1_ref, y2_ref, dinv_ref, b2_ref, fcw_ref, fcb_ref,
             out_ref, accum):
        i = pl.program_id(0)

        @pl.when(i == 0)
        def _():
            accum[...] = jnp.zeros_like(accum)

        ssum = a0_ref[...] + a1_ref[...] + y2_ref[...]
        h2 = jnp.maximum(dinv_ref[...] * ssum + b2_ref[...], 0.0)
        rid = lax.broadcasted_iota(jnp.int32, (BR, 16), 0) + i * BR
        h2 = jnp.where(rid < N, h2, 0.0)
        accum[...] += jnp.sum(h2, axis=0, keepdims=True)

        @pl.when(i == NG - 1)
        def _():
            m = accum[...] * (1.0 / N)
            logits = jnp.dot(m, fcw_ref[...],
                             preferred_element_type=jnp.float32) + fcb_ref[...]
            z = logits - jnp.max(logits, axis=1, keepdims=True)
            out_ref[...] = z - jnp.log(
                jnp.sum(jnp.exp(z), axis=1, keepdims=True))

    return pl.pallas_call(
        body,
        grid=(NG,),
        in_specs=[
            pl.BlockSpec((BR, 16), lambda i: (i, 0)),
            pl.BlockSpec((BR, 16), lambda i: (i, 0)),
            pl.BlockSpec((BR, 16), lambda i: (i, 0)),
            pl.BlockSpec((BR, 1), lambda i: (i, 0)),
            pl.BlockSpec((1, 16), lambda i: (0, 0)),
            pl.BlockSpec((16, fc_W.shape[1]), lambda i: (0, 0)),
            pl.BlockSpec((1, fc_W.shape[1]), lambda i: (0, 0)),
        ],
        out_specs=pl.BlockSpec((1, fc_W.shape[1]), lambda i: (0, 0)),
        out_shape=jax.ShapeDtypeStruct((1, fc_W.shape[1]), jnp.float32),
        scratch_shapes=[pltpu.VMEM((1, 16), jnp.float32)],
    )(acc0, acc1, y2, dinv, b2, fc_W, fc_b)


def kernel(x, edge_index, W1, b1, W2, b2, fc_W, fc_b):
    N, E = x.shape[0], edge_index.shape[1]
    NPAD = ((N + _NS * 16 - 1) // (_NS * 16)) * (_NS * 16)

    deg_call = _make_deg(E, NPAD)
    scat_call = _make_scatter(E, NPAD, 16)

    e_src = edge_index[0]
    e_dst = edge_index[1]
    cnt_a, cnt_b = deg_call(e_dst)
    cnt0 = cnt_a.reshape(NPAD, 1)
    cnt1 = cnt_b.reshape(NPAD, 1)
    xp = jnp.concatenate(
        [x, jnp.zeros((NPAD - N, x.shape[1]), x.dtype)], axis=0)
    b1r = b1.reshape(1, 16)
    b2r = b2.reshape(1, 16)
    fcbr = fc_b.reshape(1, fc_b.shape[0])

    y1, dinv = _t1(cnt0, cnt1, xp, W1, NPAD)
    s1a, s1b = scat_call(e_src, e_dst, y1)
    y2 = _t2(s1a, s1b, y1, dinv, b1r, W2, NPAD)
    s2a, s2b = scat_call(e_src, e_dst, y2)
    return _t3(s2a, s2b, y2, dinv, b2r, fc_W, fcbr, N)
